# baseline (speedup 1.0000x reference)
import functools

import jax
import jax.numpy as jnp
from jax import lax
from jax.experimental import pallas as pl
from jax.experimental.pallas import tpu as pltpu
from jax.experimental.pallas import tpu_sc as plsc

EMBED_D = 32
PACK = 128 // EMBED_D
B_TOTAL = 4096 * 50
NUM_CORES = 2
NUM_SUBCORES = 16
NW = NUM_CORES * NUM_SUBCORES
B_PER_W = B_TOTAL // NW
CHUNK = 160
N_CHUNKS = B_PER_W // CHUNK
LANES = 16

_mesh = plsc.VectorSubcoreMesh(core_axis_name="c", subcore_axis_name="s")


@functools.partial(
    pl.kernel,
    mesh=_mesh,
    out_type=jax.ShapeDtypeStruct((B_TOTAL, EMBED_D), jnp.float32),
    scratch_types=[
        pltpu.VMEM((B_PER_W,), jnp.int32),
        pltpu.VMEM((B_PER_W,), jnp.int32),
        pltpu.VMEM((CHUNK, 128), jnp.float32),
        pltpu.VMEM((CHUNK, 128), jnp.float32),
        pltpu.VMEM((CHUNK, EMBED_D), jnp.float32),
        pltpu.VMEM((CHUNK, EMBED_D), jnp.float32),
        pltpu.SemaphoreType.DMA,
        pltpu.SemaphoreType.DMA,
        pltpu.SemaphoreType.DMA,
        pltpu.SemaphoreType.DMA,
    ],
    compiler_params=pltpu.CompilerParams(
        use_tc_tiling_on_sc=True, needs_layout_passes=False),
)
def _gather_kernel(idx_hbm, tab_hbm, out_hbm, idx_v, grp_v, rows_a, rows_b,
                   ext_a, ext_b, ga_sem, gb_sem, wa_sem, wb_sem):
    wid = lax.axis_index("s") * NUM_CORES + lax.axis_index("c")
    base = wid * B_PER_W
    pltpu.sync_copy(idx_hbm.at[pl.ds(base, B_PER_W)], idx_v)

    def grp_body(i, _):
        sl = pl.ds(i * LANES, LANES)
        grp_v[sl] = lax.shift_right_logical(idx_v[sl], 2)
        return _
    lax.fori_loop(0, B_PER_W // LANES, grp_body, 0)

    rows = (rows_a, rows_b)
    ext = (ext_a, ext_b)
    gsem = (ga_sem, gb_sem)
    wsem = (wa_sem, wb_sem)

    def gather(c, p):
        pltpu.async_copy(
            tab_hbm.at[grp_v.at[pl.ds(c * CHUNK, CHUNK)]], rows[p], gsem[p])

    def drain_gather(p):
        pltpu.make_async_copy(tab_hbm.at[pl.ds(0, CHUNK)], rows[p],
                              gsem[p]).wait()

    def extract(c, p):
        def m_body(m, _):
            vec = idx_v[pl.ds(c * CHUNK + m * LANES, LANES)]
            sub = (vec & (PACK - 1)) * EMBED_D
            for l in range(LANES):
                j = m * LANES + l
                c0 = sub[l]
                ext[p][j, pl.ds(0, 16)] = rows[p][j, pl.ds(c0, 16)]
                ext[p][j, pl.ds(16, 16)] = rows[p][j, pl.ds(c0 + 16, 16)]
            return _

        lax.fori_loop(0, CHUNK // LANES, m_body, 0)

    def put(c, p):
        pltpu.async_copy(
            ext[p], out_hbm.at[pl.ds(base + c * CHUNK, CHUNK)], wsem[p])

    def drain_put(p):
        pltpu.make_async_copy(ext[p], out_hbm.at[pl.ds(base, CHUNK)],
                              wsem[p]).wait()

    def half(c, p, last):
        @pl.when(c + 1 < N_CHUNKS)
        def _():
            gather(c + 1, 1 - p)
        drain_gather(p)
        @pl.when(c >= 2)
        def _():
            drain_put(p)
        extract(c, p)
        put(c, p)

    gather(0, 0)

    def pair_body(i, _):
        half(2 * i, 0, False)
        half(2 * i + 1, 1, False)
        return _

    lax.fori_loop(0, N_CHUNKS // 2, pair_body, 0)
    drain_put(0)
    drain_put(1)


def kernel(x, wordmat):
    idx = x.reshape(-1).astype(jnp.int32)
    tab = wordmat.reshape(wordmat.shape[0] // PACK, 128)
    out = _gather_kernel(idx, tab)
    return out.reshape(x.shape + (EMBED_D,))


# speedup vs baseline: 1.7432x; 1.7432x over previous
import functools

import jax
import jax.numpy as jnp
from jax import lax
from jax.experimental import pallas as pl
from jax.experimental.pallas import tpu as pltpu
from jax.experimental.pallas import tpu_sc as plsc

EMBED_D = 32
SEQ = 50
BATCH = 4096
B_TOTAL = BATCH * SEQ
NUM_CORES = 2
NUM_SUBCORES = 16
NW = NUM_CORES * NUM_SUBCORES
B_PER_W = B_TOTAL // NW
NB = BATCH // NW
BCHUNK = 8
N_CHUNKS = NB // BCHUNK
LOOKUPS = BCHUNK * SEQ
LANES = 16

_mesh = plsc.VectorSubcoreMesh(core_axis_name="c", subcore_axis_name="s")


@functools.partial(
    pl.kernel,
    mesh=_mesh,
    out_type=jax.ShapeDtypeStruct((BATCH, SEQ, EMBED_D), jnp.float32),
    scratch_types=[
        pltpu.VMEM((B_PER_W,), jnp.int32),
        pltpu.VMEM((LOOKUPS, EMBED_D), jnp.float32),
        pltpu.VMEM((LOOKUPS, EMBED_D), jnp.float32),
        pltpu.SemaphoreType.DMA,
        pltpu.SemaphoreType.DMA,
        pltpu.SemaphoreType.DMA,
    ],
    compiler_params=pltpu.CompilerParams(
        use_tc_tiling_on_sc=True, needs_layout_passes=False),
)
def _gather_kernel(idx_hbm, tab_hbm, out_hbm, idx_v, buf_a, buf_b,
                   g_sem, wa_sem, wb_sem):
    wid = lax.axis_index("s") * NUM_CORES + lax.axis_index("c")
    base = wid * B_PER_W
    b0 = wid * NB
    pltpu.sync_copy(idx_hbm.at[pl.ds(base, B_PER_W)], idx_v)

    bufs = (buf_a, buf_b)
    wsem = (wa_sem, wb_sem)

    def fill(c, buf):
        def b_body(bb, _):
            f0 = (c * BCHUNK + bb) * SEQ

            def row_dma(r, s):
                pltpu.async_copy(tab_hbm.at[pl.ds(r, 1)],
                                 buf.at[pl.ds(bb * SEQ + s, 1)], g_sem)

            vec0 = idx_v[pl.ds(f0, LANES)]
            vec1 = idx_v[pl.ds(f0 + 16, LANES)]
            vec2 = idx_v[pl.ds(f0 + 32, LANES)]
            vec3 = idx_v[pl.ds(f0 + SEQ - LANES, LANES)]
            for l in range(LANES):
                row_dma(vec0[l], l)
            for l in range(LANES):
                row_dma(vec1[l], 16 + l)
            for l in range(2):
                row_dma(vec2[l], 32 + l)
            for l in range(LANES):
                row_dma(vec3[l], SEQ - LANES + l)
            return _

        lax.fori_loop(0, BCHUNK, b_body, 0)
        pltpu.make_async_copy(tab_hbm.at[pl.ds(0, LOOKUPS)], buf,
                              g_sem).wait()

    def put(c, buf, sem):
        def w_body(bb, _):
            b = b0 + c * BCHUNK + bb
            pltpu.async_copy(buf.at[pl.ds(bb * SEQ, SEQ)], out_hbm.at[b], sem)
            return _
        lax.fori_loop(0, BCHUNK, w_body, 0)

    def drain_put(c, buf, sem):
        def d_body(bb, _):
            pltpu.make_async_copy(buf.at[pl.ds(0, SEQ)], out_hbm.at[b0],
                                  sem).wait()
            return _
        lax.fori_loop(0, BCHUNK, d_body, 0)

    pending = [None, None]
    for c in range(N_CHUNKS):
        p = c % 2
        if pending[p] is not None:
            drain_put(pending[p], bufs[p], wsem[p])
            pending[p] = None
        fill(c, bufs[p])
        put(c, bufs[p], wsem[p])
        pending[p] = c
    for p in range(2):
        if pending[p] is not None:
            drain_put(pending[p], bufs[p], wsem[p])


def kernel(x, wordmat):
    idx = x.reshape(-1).astype(jnp.int32)
    return _gather_kernel(idx, wordmat)
